# baseline (speedup 1.0000x reference)
import functools

import jax
import jax.numpy as jnp
from jax import lax
from jax.experimental import pallas as pl
from jax.experimental.pallas import tpu as pltpu
from jax.experimental.pallas import tpu_sc as plsc

_D = 64
_M = 4
_LANES = 16

_GDN = lax.GatherDimensionNumbers(
    offset_dims=(), collapsed_slice_dims=(0,), start_index_map=(0,))


def _xlane(v, perm):
    return lax.gather(v, perm[:, None], _GDN, (1,),
                      mode=lax.GatherScatterMode.PROMISE_IN_BOUNDS)


@functools.cache
def _make_sc_call(B, S, n_opc, n_opr, interpret=False):
    try:
        info = plsc.get_sparse_core_info()
        NC, NS = info.num_cores, info.num_subcores
    except ValueError:
        NC, NS = 2, 16
    NW = NC * NS
    N = B * S
    assert B % NW == 0 and S % 8 == 0
    CH = S
    CM = CH * _M

    def _pieces(total):
        return [(st, min(128, total - st)) for st in range(0, total, 128)]
    rows_per_w = B // NW
    n_chunks = rows_per_w

    mesh = plsc.VectorSubcoreMesh(
        core_axis_name="c", subcore_axis_name="s",
        num_cores=NC, num_subcores=NS)

    @functools.partial(
        pl.kernel,
        out_type=jax.ShapeDtypeStruct((N, _D), jnp.float32),
        mesh=mesh,
        interpret=interpret,
        compiler_params=pltpu.CompilerParams(
            use_tc_tiling_on_sc=False, needs_layout_passes=False),
        scratch_types=[
            pltpu.VMEM((CH,), jnp.int32),
            pltpu.VMEM((CH, _M), jnp.int32),
            pltpu.VMEM((CM,), jnp.int32),
            pltpu.VMEM((CM,), jnp.float32),
            pltpu.VMEM((CM, _D), jnp.float32),
            pltpu.VMEM((CH, _D), jnp.float32),
            pltpu.SemaphoreType.DMA,
            pltpu.SemaphoreType.DMA,
        ],
    )
    def sc_fn(opc_ids_hbm, opr_ids_hbm, opc_tab_hbm, opr_tab_hbm, out_hbm,
              opc_v, ids_v, ids_f, w_v, rows_v, o_v, sem_ids, sem_g):
        wid = lax.axis_index("s") * NC + lax.axis_index("c")
        b0 = wid * rows_per_w
        lane = lax.iota(jnp.int32, _LANES)
        perm1 = lane ^ 1
        perm2 = lane ^ 2
        colv = lane & (_M - 1)

        def chunk_body(k, carry):
            b = b0 + k
            base = b * S
            cp0 = pltpu.async_copy(opc_ids_hbm.at[b], opc_v, sem_ids)
            cp1 = pltpu.async_copy(opr_ids_hbm.at[b], ids_v, sem_ids)
            cp0.wait()
            cp1.wait()
            gs = [pltpu.async_copy(
                      opc_tab_hbm.at[opc_v.at[pl.ds(st, ln)]],
                      o_v.at[pl.ds(st, ln)], sem_g)
                  for st, ln in _pieces(CH)]
            for t in range(CM // _LANES):
                rowv = (lane + t * _LANES) >> 2
                idv = plsc.load_gather(ids_v, [rowv, colv])
                ids_f[pl.ds(t * _LANES, _LANES)] = idv
                mk = jnp.where(idv != 0, 1.0, 0.0)
                s = mk + _xlane(mk, perm1)
                s = s + _xlane(s, perm2)
                w_v[pl.ds(t * _LANES, _LANES)] = mk / (s + 1e-10)
            for st, ln in _pieces(CM):
                gs.append(pltpu.async_copy(
                    opr_tab_hbm.at[ids_f.at[pl.ds(st, ln)]],
                    rows_v.at[pl.ds(st, ln)], sem_g))
            for g in gs:
                g.wait()

            def group_body(g, carry2):
                r0 = g * _LANES
                i0 = g * (_LANES // _M)
                wvec = w_v[pl.ds(r0, _LANES)]
                for j in range(_LANES // _M):
                    for dblk in range(_D // _LANES):
                        sl = pl.ds(dblk * _LANES, _LANES)
                        acc = wvec[4 * j] * rows_v[r0 + 4 * j, sl]
                        for m in range(1, _M):
                            acc = acc + wvec[4 * j + m] * rows_v[r0 + 4 * j + m, sl]
                        plsc.addupdate(o_v.at[i0 + j, sl], acc)
                return carry2

            lax.fori_loop(0, CM // _LANES, group_body, 0)
            pltpu.sync_copy(o_v, out_hbm.at[pl.ds(base, CH)])
            return carry

        lax.fori_loop(0, n_chunks, chunk_body, 0)

    return sc_fn


def kernel(opcode_ids, operand_ids, opcode_table, operand_table):
    B, S = opcode_ids.shape
    fn = _make_sc_call(B, S, opcode_table.shape[0], operand_table.shape[0])
    out = fn(opcode_ids.astype(jnp.int32), operand_ids.astype(jnp.int32),
             opcode_table, operand_table)
    return out.reshape(B, S, _D)


# speedup vs baseline: 1.0997x; 1.0997x over previous
import functools

import jax
import jax.numpy as jnp
from jax import lax
from jax.experimental import pallas as pl
from jax.experimental.pallas import tpu as pltpu
from jax.experimental.pallas import tpu_sc as plsc

_D = 64
_M = 4
_LANES = 16

_GDN = lax.GatherDimensionNumbers(
    offset_dims=(), collapsed_slice_dims=(0,), start_index_map=(0,))


def _xlane(v, perm):
    return lax.gather(v, perm[:, None], _GDN, (1,),
                      mode=lax.GatherScatterMode.PROMISE_IN_BOUNDS)


@functools.cache
def _make_sc_call(B, S, n_opc, n_opr, interpret=False):
    try:
        info = plsc.get_sparse_core_info()
        NC, NS = info.num_cores, info.num_subcores
    except ValueError:
        NC, NS = 2, 16
    NW = NC * NS
    N = B * S
    assert B % NW == 0 and S % 8 == 0
    CH = S
    CM = CH * _M

    def _pieces(total):
        return [(st, min(128, total - st)) for st in range(0, total, 128)]
    rows_per_w = B // NW
    n_chunks = rows_per_w

    mesh = plsc.VectorSubcoreMesh(
        core_axis_name="c", subcore_axis_name="s",
        num_cores=NC, num_subcores=NS)

    @functools.partial(
        pl.kernel,
        out_type=jax.ShapeDtypeStruct((N, _D), jnp.float32),
        mesh=mesh,
        interpret=interpret,
        compiler_params=pltpu.CompilerParams(
            use_tc_tiling_on_sc=False, needs_layout_passes=False),
        scratch_types=[
            pltpu.VMEM((CH, rows_per_w), jnp.int32),
            pltpu.VMEM((CH, _M, rows_per_w), jnp.int32),
            pltpu.VMEM((-(-CH // _LANES) * _LANES,), jnp.int32),
            pltpu.VMEM((CM,), jnp.int32),
            pltpu.VMEM((CM,), jnp.float32),
            pltpu.VMEM((CM, _D), jnp.float32),
            pltpu.VMEM((CH, _D), jnp.float32),
            pltpu.SemaphoreType.DMA,
            pltpu.SemaphoreType.DMA,
        ],
    )
    def sc_fn(opc_ids_hbm, opr_ids_hbm, opc_tab_hbm, opr_tab_hbm, out_hbm,
              opcb_v, idsb_v, opc_f, ids_f, w_v, rows_v, o_v, sem_ids, sem_g):
        wid = lax.axis_index("s") * NC + lax.axis_index("c")
        b0 = wid * rows_per_w
        lane = lax.iota(jnp.int32, _LANES)
        perm1 = lane ^ 1
        perm2 = lane ^ 2
        colv = lane & (_M - 1)

        pltpu.async_copy(
            opc_ids_hbm.at[:, pl.ds(b0, rows_per_w)], opcb_v, sem_ids).wait()
        pltpu.async_copy(
            opr_ids_hbm.at[:, :, pl.ds(b0, rows_per_w)], idsb_v,
            sem_ids).wait()

        def chunk_body(k, carry):
            b = b0 + k
            base = b * S
            for t in range(CH // _LANES + (1 if CH % _LANES else 0)):
                sv = jnp.minimum(lane + t * _LANES, CH - 1)
                opc_f[pl.ds(t * _LANES, _LANES)] = plsc.load_gather(
                    opcb_v, [sv, jnp.broadcast_to(k, (_LANES,))])
            for t in range(CM // _LANES):
                p = lane + t * _LANES
                idv = plsc.load_gather(
                    idsb_v, [p >> 2, colv, jnp.broadcast_to(k, (_LANES,))])
                ids_f[pl.ds(t * _LANES, _LANES)] = idv
                mk = jnp.where(idv != 0, 1.0, 0.0)
                s = mk + _xlane(mk, perm1)
                s = s + _xlane(s, perm2)
                w_v[pl.ds(t * _LANES, _LANES)] = mk / (s + 1e-10)
            gs = [pltpu.async_copy(
                      opc_tab_hbm.at[opc_f.at[pl.ds(st, ln)]],
                      o_v.at[pl.ds(st, ln)], sem_g)
                  for st, ln in _pieces(CH)]
            for st, ln in _pieces(CM):
                gs.append(pltpu.async_copy(
                    opr_tab_hbm.at[ids_f.at[pl.ds(st, ln)]],
                    rows_v.at[pl.ds(st, ln)], sem_g))
            for g in gs:
                g.wait()

            def group_body(g, carry2):
                r0 = g * _LANES
                i0 = g * (_LANES // _M)
                wvec = w_v[pl.ds(r0, _LANES)]
                for j in range(_LANES // _M):
                    for dblk in range(_D // _LANES):
                        sl = pl.ds(dblk * _LANES, _LANES)
                        acc = wvec[4 * j] * rows_v[r0 + 4 * j, sl]
                        for m in range(1, _M):
                            acc = acc + wvec[4 * j + m] * rows_v[r0 + 4 * j + m, sl]
                        plsc.addupdate(o_v.at[i0 + j, sl], acc)
                return carry2

            lax.fori_loop(0, CM // _LANES, group_body, 0)
            pltpu.sync_copy(o_v, out_hbm.at[pl.ds(base, CH)])
            return carry

        lax.fori_loop(0, n_chunks, chunk_body, 0)

    return sc_fn


def kernel(opcode_ids, operand_ids, opcode_table, operand_table):
    B, S = opcode_ids.shape
    fn = _make_sc_call(B, S, opcode_table.shape[0], operand_table.shape[0])
    opc_t = opcode_ids.T.astype(jnp.int32)
    opr_t = jnp.transpose(operand_ids, (1, 2, 0)).astype(jnp.int32)
    out = fn(opc_t, opr_t, opcode_table, operand_table)
    return out.reshape(B, S, _D)


# speedup vs baseline: 1.1262x; 1.0241x over previous
import functools

import jax
import jax.numpy as jnp
from jax import lax
from jax.experimental import pallas as pl
from jax.experimental.pallas import tpu as pltpu
from jax.experimental.pallas import tpu_sc as plsc

_D = 64
_M = 4
_LANES = 16
_SB = 4
_BB = 32


@functools.cache
def _make_sc_call(B, S, n_opc, n_opr, interpret=False):
    try:
        info = plsc.get_sparse_core_info()
        NC, NS = info.num_cores, info.num_subcores
    except ValueError:
        NC, NS = 2, 16
    NW = NC * NS
    N = B * S
    assert B % (NW * _BB) == 0 or B == NW * _BB
    assert S % _SB == 0
    n_chunks = S // _SB
    CR = _SB * _BB
    CM = CR * _M

    mesh = plsc.VectorSubcoreMesh(
        core_axis_name="c", subcore_axis_name="s",
        num_cores=NC, num_subcores=NS)

    @functools.partial(
        pl.kernel,
        out_type=jax.ShapeDtypeStruct((N, _D), jnp.float32),
        mesh=mesh,
        interpret=interpret,
        compiler_params=pltpu.CompilerParams(use_tc_tiling_on_sc=False),
        scratch_types=[
            pltpu.VMEM((S, _BB), jnp.int32),
            pltpu.VMEM((S, _M, _BB), jnp.int32),
            pltpu.VMEM((CR,), jnp.int32),
            pltpu.VMEM((CM,), jnp.int32),
            pltpu.VMEM((CM,), jnp.float32),
            pltpu.VMEM((CM, _D), jnp.float32),
            pltpu.VMEM((CR, _D), jnp.float32),
            pltpu.SemaphoreType.DMA,
            pltpu.SemaphoreType.DMA,
        ],
    )
    def sc_fn(opc_ids_hbm, opr_ids_hbm, opc_tab_hbm, opr_tab_hbm, out_hbm,
              opcb_v, idsb_v, opc_f, ids_f, w_v, rows_v, o_v, sem_ids, sem_g):
        wid = lax.axis_index("s") * NC + lax.axis_index("c")
        b0 = wid * _BB

        pltpu.async_copy(
            opc_ids_hbm.at[:, pl.ds(b0, _BB)], opcb_v, sem_ids).wait()
        pltpu.async_copy(
            opr_ids_hbm.at[:, :, pl.ds(b0, _BB)], idsb_v, sem_ids).wait()

        def chunk_body(k, carry):
            s0 = k * _SB
            for si in range(_SB):
                for h in range(_BB // _LANES):
                    sl = pl.ds(h * _LANES, _LANES)
                    opc_f[pl.ds(si * _BB + h * _LANES, _LANES)] = (
                        opcb_v[s0 + si, sl])
                    idv = [idsb_v[s0 + si, m, sl] for m in range(_M)]
                    mk = [jnp.where(v != 0, 1.0, 0.0) for v in idv]
                    cnt = mk[0] + mk[1] + mk[2] + mk[3] + 1e-10
                    for m in range(_M):
                        off = (si * _M + m) * _BB + h * _LANES
                        ids_f[pl.ds(off, _LANES)] = idv[m]
                        w_v[pl.ds(off, _LANES)] = mk[m] / cnt
            gs = [pltpu.async_copy(opc_tab_hbm.at[opc_f], o_v, sem_g)]
            for q in range(_SB):
                gs.append(pltpu.async_copy(
                    opr_tab_hbm.at[ids_f.at[pl.ds(q * _M * _BB, _M * _BB)]],
                    rows_v.at[pl.ds(q * _M * _BB, _M * _BB)], sem_g))
            for g in gs:
                g.wait()

            def acc_body(t, carry2):
                si = t >> 1
                h = t & 1
                wvecs = [w_v[pl.ds((si * _M + m) * _BB + h * _LANES, _LANES)]
                         for m in range(_M)]
                for j in range(_LANES):
                    r = si * _M * _BB + h * _LANES + j
                    o_r = si * _BB + h * _LANES + j
                    for dblk in range(_D // _LANES):
                        sl = pl.ds(dblk * _LANES, _LANES)
                        acc = wvecs[0][j] * rows_v[r, sl]
                        for m in range(1, _M):
                            acc = acc + wvecs[m][j] * rows_v[r + m * _BB, sl]
                        plsc.addupdate(o_v.at[o_r, sl], acc)
                return carry2

            lax.fori_loop(0, _SB * (_BB // _LANES), acc_body, 0)
            for si in range(_SB):
                pltpu.async_copy(
                    o_v.at[pl.ds(si * _BB, _BB)],
                    out_hbm.at[pl.ds((s0 + si) * B + b0, _BB)],
                    sem_ids).wait()
            return carry

        lax.fori_loop(0, n_chunks, chunk_body, 0)

    return sc_fn


def kernel(opcode_ids, operand_ids, opcode_table, operand_table):
    B, S = opcode_ids.shape
    fn = _make_sc_call(B, S, opcode_table.shape[0], operand_table.shape[0])
    opc_t = opcode_ids.T.astype(jnp.int32)
    opr_t = jnp.transpose(operand_ids, (1, 2, 0)).astype(jnp.int32)
    out = fn(opc_t, opr_t, opcode_table, operand_table)
    return jnp.transpose(out.reshape(S, B, _D), (1, 0, 2))
